# baseline (speedup 1.0000x reference)
import jax
import jax.numpy as jnp
from jax import lax
from jax.experimental import pallas as pl
from jax.experimental.pallas import tpu as pltpu
from jax.experimental.pallas import tpu_sc as plsc

_N = 10000
_E = 320000
_D = 128
_DH = 64
_L = 16
_NS = 16
_K = 128
_CHUNKS = 160
_PAIRS = _CHUNKS // 2
_EPT = _CHUNKS * _K
_EPAD = _EPT * _NS
_TOTCH = _EPAD // _K
_TOTCH_PAD = _TOTCH + 4
_RPT = _N // _NS


def _mm_body(x_ref, w_ref, o0_ref, o1_ref):
    s = jnp.dot(x_ref[...], w_ref[...], preferred_element_type=jnp.float32)
    o0_ref[...] = s[:, :_DH]
    o1_ref[...] = s[:, _DH:]


def _sc_body(sup0, sup1, ed, bias_hbm, out_hbm,
             edv0, edv1, rows0, rows1, bias_v, wb_v, acc_sh, sup_sh,
             semi0, semi1, semg0, semg1):
    c = lax.axis_index("c")
    s = lax.axis_index("s")
    edv = (edv0, edv1)
    rows_v = (rows0, rows1)
    semi = (semi0, semi1)
    semg = (semg0, semg1)

    pltpu.sync_copy(bias_hbm.at[pl.ds(c * _DH, _DH)], bias_v)

    def init_body(r, carry):
        for j in range(_DH // _L):
            wb_v[r, pl.ds(j * _L, _L)] = bias_v[pl.ds(j * _L, _L)]
        return carry

    lax.fori_loop(0, _RPT // 5, init_body, None)
    for t in range(5):
        pltpu.sync_copy(wb_v, acc_sh.at[pl.ds(s * _RPT + t * (_RPT // 5), _RPT // 5)])

    @pl.when(c == 0)
    def _stage0():
        pltpu.sync_copy(sup0.at[pl.ds(s * _RPT, _RPT)],
                        sup_sh.at[pl.ds(s * _RPT, _RPT)])

    @pl.when(c == 1)
    def _stage1():
        pltpu.sync_copy(sup1.at[pl.ds(s * _RPT, _RPT)],
                        sup_sh.at[pl.ds(s * _RPT, _RPT)])

    plsc.subcore_barrier()

    pair0 = s * _PAIRS

    def issue_fetch(pj, e):
        pltpu.async_copy(ed.at[pl.ds((pair0 + pj) * 2, 2)], edv[e], semi[e])

    def wait_fetch(e):
        pltpu.make_async_copy(ed.at[pl.ds(0, 2)], edv[e], semi[e]).wait()

    def issue_gather(e, q, b):
        pltpu.async_copy(sup_sh.at[edv[e].at[q, 0]], rows_v[b], semg[b])

    def wait_gather(e, q, b):
        pltpu.make_async_copy(sup_sh.at[edv[e].at[q, 0]], rows_v[b],
                              semg[b]).wait()

    def scale(e, q, b):
        def scale_body(g, carry):
            vv = plsc.bitcast(edv[e][q, 1, pl.ds(g * _L, _L)], jnp.float32)
            for k in range(_L):
                r = g * _L + k
                v = vv[k]
                for j in range(_DH // _L):
                    rows_v[b][r, pl.ds(j * _L, _L)] = (
                        rows_v[b][r, pl.ds(j * _L, _L)] * v)
            return carry

        lax.fori_loop(0, _K // _L, scale_body, None)

    def scatter(e, q, b):
        pltpu.sync_copy(rows_v[b], acc_sh.at[edv[e].at[q, 2]], add=True)

    issue_fetch(0, 0)
    wait_fetch(0)
    issue_gather(0, 0, 0)
    issue_fetch(1, 1)

    def quad_body(qq, carry):
        for p in (0, 1):
            pj = 2 * qq + p
            wait_gather(p, 0, 0)
            wait_fetch(1 - p)
            issue_gather(p, 1, 1)
            scale(p, 0, 0)
            scatter(p, 0, 0)
            wait_gather(p, 1, 1)
            issue_gather(1 - p, 0, 0)
            scale(p, 1, 1)
            scatter(p, 1, 1)
            issue_fetch(pj + 2, p)
        return carry

    lax.fori_loop(0, _PAIRS // 2, quad_body, None)
    wait_gather(0, 0, 0)
    wait_fetch(1)

    plsc.subcore_barrier()
    pltpu.sync_copy(acc_sh.at[pl.ds(s * _RPT, _RPT)],
                    out_hbm.at[pl.ds(s * _RPT, _RPT), pl.ds(c * _DH, _DH)])


def kernel(x, adj_indices, adj_values, weight, bias):
    nb = 10
    support0, support1 = pl.pallas_call(
        _mm_body,
        grid=(nb,),
        in_specs=[
            pl.BlockSpec((_N // nb, _D), lambda i: (i, 0)),
            pl.BlockSpec((_D, _D), lambda i: (0, 0)),
        ],
        out_specs=[
            pl.BlockSpec((_N // nb, _DH), lambda i: (i, 0)),
            pl.BlockSpec((_N // nb, _DH), lambda i: (i, 0)),
        ],
        out_shape=[
            jax.ShapeDtypeStruct((_N, _DH), jnp.float32),
            jax.ShapeDtypeStruct((_N, _DH), jnp.float32),
        ],
    )(x, weight)

    row = adj_indices[0].astype(jnp.int32)
    col = adj_indices[1].astype(jnp.int32)
    val = jax.lax.bitcast_convert_type(adj_values.astype(jnp.float32),
                                       jnp.int32)
    pad = _TOTCH_PAD * _K - _E
    row = jnp.concatenate([row, jnp.zeros((pad,), jnp.int32)])
    col = jnp.concatenate([col, jnp.zeros((pad,), jnp.int32)])
    val = jnp.concatenate([val, jnp.zeros((pad,), jnp.int32)])
    ed = jnp.stack([col.reshape(_TOTCH_PAD, _K),
                    val.reshape(_TOTCH_PAD, _K),
                    row.reshape(_TOTCH_PAD, _K)], axis=1)

    mesh = plsc.VectorSubcoreMesh(core_axis_name="c", subcore_axis_name="s")
    sc = pl.kernel(
        _sc_body,
        mesh=mesh,
        compiler_params=pltpu.CompilerParams(use_tc_tiling_on_sc=False,
                                             needs_layout_passes=False),
        out_type=jax.ShapeDtypeStruct((_N, _D), jnp.float32),
        scratch_types=[
            pltpu.VMEM((2, 3, _K), jnp.int32),
            pltpu.VMEM((2, 3, _K), jnp.int32),
            pltpu.VMEM((_K, _DH), jnp.float32),
            pltpu.VMEM((_K, _DH), jnp.float32),
            pltpu.VMEM((_DH,), jnp.float32),
            pltpu.VMEM((_RPT // 5, _DH), jnp.float32),
            pltpu.VMEM_SHARED((_N, _DH), jnp.float32),
            pltpu.VMEM_SHARED((_N, _DH), jnp.float32),
            pltpu.SemaphoreType.DMA,
            pltpu.SemaphoreType.DMA,
            pltpu.SemaphoreType.DMA,
            pltpu.SemaphoreType.DMA,
        ],
    )
    return sc(support0, support1, ed, bias)
